# baseline (speedup 1.0000x reference)
import jax
import jax.numpy as jnp
from jax.experimental import pallas as pl
from jax.experimental.pallas import tpu as pltpu

D = 256
TEMPERATURE = 10.0
BLOCK = 2048

_NT = (((1,), (1,)), ((), ()))


def _dnt(a, b):
    return jax.lax.dot_general(a, b, dimension_numbers=_NT,
                               preferred_element_type=jnp.float32)


def _gelu_exact(x):
    return 0.5 * x * (1.0 + jax.lax.erf(x * 0.7071067811865476))


def _ln(x, g, b, eps=1e-5):
    m = jnp.mean(x, axis=-1, keepdims=True)
    c = x - m
    v = jnp.mean(c * c, axis=-1, keepdims=True)
    return c * jax.lax.rsqrt(v + eps) * g + b


def _kernel_body(xt_ref, xi_ref, g_ref,
                 wvi_ref, wot_ref, wvt_ref, woi_ref,
                 c0_ref, sh0_ref, c1_ref, n0_ref, n1_ref,
                 c2_ref, m2_ref, sh1_ref, pk_ref,
                 logits_ref, routing_ref, normal_ref, conflict_ref,
                 sarcasm_ref,
                 s_xtw, s_xiw, s_bt, s_bi, s_ua, s_ba,
                 s_wt, s_wi, s_c1, s_n1):
    f32 = jnp.float32
    bf16 = jnp.bfloat16

    @pl.when(pl.program_id(0) == 0)
    def _fold():
        wot = wot_ref[:].astype(bf16)
        wvi = wvi_ref[:].astype(bf16)
        woi = woi_ref[:].astype(bf16)
        wvt = wvt_ref[:].astype(bf16)
        s_xiw[0:D, :] = jnp.dot(wot, wvi, preferred_element_type=f32).astype(bf16)
        s_xiw[D:, :] = n0_ref[:, D:].astype(bf16)
        s_xtw[0:D, :] = jnp.dot(woi, wvt, preferred_element_type=f32).astype(bf16)
        s_xtw[D:, :] = n0_ref[:, :D].astype(bf16)
        s_bt[:] = _dnt(pk_ref[0:1, :], wot_ref[:]) + pk_ref[1:2, :]
        s_bi[:] = _dnt(pk_ref[2:3, :], woi_ref[:]) + pk_ref[3:4, :]
        wgc = c0_ref[:, 2 * D:2 * D + 32]
        wgs = sh0_ref[:, 2 * D:2 * D + 32]
        s_ua[:, 0:D] = _dnt(pk_ref[16:17, 0:32], wgc)
        s_ua[:, D:] = _dnt(pk_ref[18:19, 0:32], wgs)
        s_ba[:, 0:D] = _dnt(pk_ref[17:18, 0:32], wgc) + pk_ref[8:9, :]
        s_ba[:, D:] = _dnt(pk_ref[19:20, 0:32], wgs) + pk_ref[9:10, 0:128]
        s_wt[0:D, :] = c0_ref[:, :D].astype(bf16)
        s_wt[D:, :] = sh0_ref[:, :D].astype(bf16)
        s_wi[0:D, :] = c0_ref[:, D:2 * D].astype(bf16)
        s_wi[D:, :] = sh0_ref[:, D:2 * D].astype(bf16)
        s_c1[:] = c1_ref[:].astype(bf16)
        s_n1[:] = n1_ref[:].astype(bf16)

    xt = xt_ref[:]
    xi = xi_ref[:]
    g2 = g_ref[:]
    gt = jnp.transpose(g2)
    g = jnp.concatenate(
        [gt[:, r:r + 1] for r in range(g2.shape[0])], axis=0)
    xtb = xt.astype(bf16)
    xib = xi.astype(bf16)
    p_t = _dnt(xtb, s_xtw[:])
    p_i = _dnt(xib, s_xiw[:])
    t_refv = _ln(xt + (p_i[:, :D] + s_bt[:]), pk_ref[4:5, :], pk_ref[5:6, :])
    i_refv = _ln(xi + (p_t[:, :D] + s_bi[:]), pk_ref[6:7, :], pk_ref[7:8, :])
    trb = t_refv.astype(bf16)
    irb = i_refv.astype(bf16)
    q = _dnt(trb, s_wt[:]) + _dnt(irb, s_wi[:]) + g * s_ua[:] + s_ba[:]
    qa = _gelu_exact(q.astype(bf16))
    h0b = qa[:, :D]
    hsb = qa[:, D:]
    h1b = _gelu_exact((_dnt(h0b, s_c1[:]) + pk_ref[10:11, 0:128]).astype(bf16))
    n_pre = p_t[:, D:] + p_i[:, D:] + pk_ref[11:12, :]
    n0b_ = _gelu_exact(n_pre.astype(bf16))
    n1b_ = _gelu_exact((_dnt(n0b_, s_n1[:]) + pk_ref[12:13, 0:128]).astype(bf16))
    conflict = _dnt(h1b, c2_ref[:].astype(bf16)) + pk_ref[13:14, 0:3]
    normal = _dnt(n1b_, m2_ref[:].astype(bf16)) + pk_ref[14:15, 0:3]
    sarcasm = _dnt(hsb, sh1_ref[:].astype(bf16)) + pk_ref[15:16, 0:2]
    tau = jax.nn.sigmoid(pk_ref[20:21, 0:1])
    gate = jax.nn.sigmoid((g - tau) * TEMPERATURE)
    logits_ref[:] = gate * conflict + (1.0 - gate) * normal
    routing_ref[:] = (g2 > tau).astype(jnp.float32)
    normal_ref[:] = normal
    conflict_ref[:] = conflict
    sarcasm_ref[:] = sarcasm


@jax.jit
def _run(s_t, s_i, gds, params):
    f32 = jnp.float32
    bf16 = jnp.bfloat16
    p = params
    B = s_t.shape[0]
    grid = (B // BLOCK,)
    row = lambda i: (i, 0)
    rep = lambda i: (0, 0)
    vhalf = lambda i: (1, 0)

    def _row_pad(v):
        v = v.astype(f32).ravel()
        n = v.shape[0]
        if n < D:
            v = jnp.concatenate([v, jnp.zeros((D - n,), f32)])
        return v[None]

    pk = jnp.concatenate([
        _row_pad(p['ca_kvpi_b'][D:]),
        _row_pad(p['ca_opt_b']),
        _row_pad(p['ca_kvpt_b'][D:]),
        _row_pad(p['ca_opi_b']),
        _row_pad(p['ca_lnt_g']),
        _row_pad(p['ca_lnt_b']),
        _row_pad(p['ca_lni_g']),
        _row_pad(p['ca_lni_b']),
        _row_pad(p['cb_c0_b']),
        _row_pad(p['sh_h0_b']),
        _row_pad(p['cb_c1_b']),
        _row_pad(p['nb_m0_b']),
        _row_pad(p['nb_m1_b']),
        _row_pad(p['cb_c2_b']),
        _row_pad(p['nb_m2_b']),
        _row_pad(p['sh_h1_b']),
        _row_pad(p['cb_gds_w']),
        _row_pad(p['cb_gds_b']),
        _row_pad(p['sh_gds_w']),
        _row_pad(p['sh_gds_b']),
        _row_pad(p['log_threshold']),
        jnp.zeros((3, D), f32),
    ], axis=0)

    in_specs = [
        pl.BlockSpec((BLOCK, D), row),
        pl.BlockSpec((BLOCK, D), row),
        pl.BlockSpec((BLOCK // 128, 128), row),
        pl.BlockSpec((D, D), vhalf),
        pl.BlockSpec((D, D), rep),
        pl.BlockSpec((D, D), vhalf),
        pl.BlockSpec((D, D), rep),
        pl.BlockSpec((D, 2 * D + 32), rep),
        pl.BlockSpec((128, 2 * D + 32), rep),
        pl.BlockSpec((128, D), rep),
        pl.BlockSpec((D, 2 * D), rep),
        pl.BlockSpec((128, D), rep),
        pl.BlockSpec((3, 128), rep),
        pl.BlockSpec((3, 128), rep),
        pl.BlockSpec((2, 128), rep),
        pl.BlockSpec((24, D), rep),
    ]
    out_specs = [
        pl.BlockSpec((BLOCK, 3), row),
        pl.BlockSpec((BLOCK // 128, 128), row),
        pl.BlockSpec((BLOCK, 3), row),
        pl.BlockSpec((BLOCK, 3), row),
        pl.BlockSpec((BLOCK, 2), row),
    ]
    out_shape = [
        jax.ShapeDtypeStruct((B, 3), f32),
        jax.ShapeDtypeStruct((B // 128, 128), f32),
        jax.ShapeDtypeStruct((B, 3), f32),
        jax.ShapeDtypeStruct((B, 3), f32),
        jax.ShapeDtypeStruct((B, 2), f32),
    ]
    scratch_shapes = [
        pltpu.VMEM((2 * D, D), bf16),
        pltpu.VMEM((2 * D, D), bf16),
        pltpu.VMEM((1, D), f32),
        pltpu.VMEM((1, D), f32),
        pltpu.VMEM((1, 384), f32),
        pltpu.VMEM((1, 384), f32),
        pltpu.VMEM((384, D), bf16),
        pltpu.VMEM((384, D), bf16),
        pltpu.VMEM((128, D), bf16),
        pltpu.VMEM((128, D), bf16),
    ]
    outs = pl.pallas_call(
        _kernel_body,
        grid=grid,
        in_specs=in_specs,
        out_specs=out_specs,
        out_shape=out_shape,
        scratch_shapes=scratch_shapes,
        compiler_params=pltpu.CompilerParams(
            dimension_semantics=("arbitrary",)),
    )(s_t, s_i, gds.reshape(B // 128, 128),
      p['ca_kvpi_w'], p['ca_opt_w'], p['ca_kvpt_w'], p['ca_opi_w'],
      p['cb_c0_w'], p['sh_h0_w'], p['cb_c1_w'],
      p['nb_m0_w'], p['nb_m1_w'],
      p['cb_c2_w'], p['nb_m2_w'], p['sh_h1_w'],
      pk)
    logits, routing, normal, conflict, sarcasm = outs
    return logits, routing.reshape(B), normal, conflict, sarcasm


def kernel(s_t, s_i, gds, params):
    return _run(s_t, s_i, gds, params)


# speedup vs baseline: 1.0436x; 1.0436x over previous
import jax
import jax.numpy as jnp
from jax.experimental import pallas as pl
from jax.experimental.pallas import tpu as pltpu

D = 256
TEMPERATURE = 10.0
BLOCK = 4096

_NT = (((1,), (1,)), ((), ()))


def _dnt(a, b):
    return jax.lax.dot_general(a, b, dimension_numbers=_NT,
                               preferred_element_type=jnp.float32)


def _gelu_exact(x):
    return 0.5 * x * (1.0 + jax.lax.erf(x * 0.7071067811865476))


def _ln(x, g, b, eps=1e-5):
    m = jnp.mean(x, axis=-1, keepdims=True)
    c = x - m
    v = jnp.mean(c * c, axis=-1, keepdims=True)
    return c * jax.lax.rsqrt(v + eps) * g + b


def _kernel_body(xt_ref, xi_ref, g_ref,
                 wvi_ref, wot_ref, wvt_ref, woi_ref,
                 bvi_ref, bot_ref, bvt_ref, boi_ref,
                 lntg_ref, lntb_ref, lnig_ref, lnib_ref,
                 c0_ref, c0b_ref, gwc_ref, gbc_ref,
                 sh0_ref, sh0b_ref, gws_ref, gbs_ref,
                 c1_ref, c1b_ref,
                 n0_ref, n0b_ref, n1_ref, n1b_ref,
                 c2_ref, c2b_ref, m2_ref, m2b_ref, sh1_ref, sh1b_ref,
                 lt_ref,
                 logits_ref, routing_ref, normal_ref, conflict_ref,
                 sarcasm_ref,
                 s_xtw, s_xiw, s_bt, s_bi, s_ua, s_ba,
                 s_wt, s_wi, s_c1, s_n1):
    f32 = jnp.float32
    bf16 = jnp.bfloat16

    @pl.when(pl.program_id(0) == 0)
    def _fold():
        wot = wot_ref[:].astype(bf16)
        wvi = wvi_ref[:].astype(bf16)
        woi = woi_ref[:].astype(bf16)
        wvt = wvt_ref[:].astype(bf16)
        s_xiw[0:D, :] = jnp.dot(wot, wvi, preferred_element_type=f32).astype(bf16)
        s_xiw[D:, :] = n0_ref[:, D:].astype(bf16)
        s_xtw[0:D, :] = jnp.dot(woi, wvt, preferred_element_type=f32).astype(bf16)
        s_xtw[D:, :] = n0_ref[:, :D].astype(bf16)
        s_bt[:] = _dnt(bvi_ref[:], wot_ref[:]) + bot_ref[:]
        s_bi[:] = _dnt(bvt_ref[:], woi_ref[:]) + boi_ref[:]
        wgc = c0_ref[:, 2 * D:2 * D + 32]
        wgs = sh0_ref[:, 2 * D:2 * D + 32]
        s_ua[:, 0:D] = jnp.reshape(
            jnp.dot(wgc, gwc_ref[:], preferred_element_type=f32), (1, D))
        s_ua[:, D:] = jnp.reshape(
            jnp.dot(wgs, gws_ref[:], preferred_element_type=f32), (1, 128))
        s_ba[:, 0:D] = _dnt(gbc_ref[:], wgc) + c0b_ref[:]
        s_ba[:, D:] = _dnt(gbs_ref[:], wgs) + sh0b_ref[:]
        s_wt[0:D, :] = c0_ref[:, :D].astype(bf16)
        s_wt[D:, :] = sh0_ref[:, :D].astype(bf16)
        s_wi[0:D, :] = c0_ref[:, D:2 * D].astype(bf16)
        s_wi[D:, :] = sh0_ref[:, D:2 * D].astype(bf16)
        s_c1[:] = c1_ref[:].astype(bf16)
        s_n1[:] = n1_ref[:].astype(bf16)

    xt = xt_ref[:]
    xi = xi_ref[:]
    g2 = g_ref[:]
    gt = jnp.transpose(g2)
    g = jnp.concatenate(
        [gt[:, r:r + 1] for r in range(g2.shape[0])], axis=0)
    xtb = xt.astype(bf16)
    xib = xi.astype(bf16)
    p_t = _dnt(xtb, s_xtw[:])
    p_i = _dnt(xib, s_xiw[:])
    t_refv = _ln(xt + (p_i[:, :D] + s_bt[:]), lntg_ref[:], lntb_ref[:])
    i_refv = _ln(xi + (p_t[:, :D] + s_bi[:]), lnig_ref[:], lnib_ref[:])
    trb = t_refv.astype(bf16)
    irb = i_refv.astype(bf16)
    q = _dnt(trb, s_wt[:]) + _dnt(irb, s_wi[:]) + g * s_ua[:] + s_ba[:]
    qa = _gelu_exact(q.astype(bf16))
    h0b = qa[:, :D]
    hsb = qa[:, D:]
    h1b = _gelu_exact((_dnt(h0b, s_c1[:]) + c1b_ref[:]).astype(bf16))
    n_pre = p_t[:, D:] + p_i[:, D:] + n0b_ref[:]
    n0b_ = _gelu_exact(n_pre.astype(bf16))
    n1b_ = _gelu_exact((_dnt(n0b_, s_n1[:]) + n1b_ref[:]).astype(bf16))
    conflict = _dnt(h1b, c2_ref[:].astype(bf16)) + c2b_ref[:]
    normal = _dnt(n1b_, m2_ref[:].astype(bf16)) + m2b_ref[:]
    sarcasm = _dnt(hsb, sh1_ref[:].astype(bf16)) + sh1b_ref[:]
    tau = jax.nn.sigmoid(lt_ref[:])
    gate = jax.nn.sigmoid((g - tau) * TEMPERATURE)
    logits_ref[:] = gate * conflict + (1.0 - gate) * normal
    routing_ref[:] = (g2 > tau).astype(f32)
    normal_ref[:] = normal
    conflict_ref[:] = conflict
    sarcasm_ref[:] = sarcasm


@jax.jit
def _run(s_t, s_i, gds, params):
    f32 = jnp.float32
    bf16 = jnp.bfloat16
    p = params
    B = s_t.shape[0]
    grid = (B // BLOCK,)
    row = lambda i: (i, 0)
    rep = lambda i: (0, 0)
    vhalf = lambda i: (1, 0)

    in_specs = [
        pl.BlockSpec((BLOCK, D), row),
        pl.BlockSpec((BLOCK, D), row),
        pl.BlockSpec((BLOCK // 128, 128), row),
        pl.BlockSpec((D, D), vhalf),
        pl.BlockSpec((D, D), rep),
        pl.BlockSpec((D, D), vhalf),
        pl.BlockSpec((D, D), rep),
        pl.BlockSpec((1, D), rep),
        pl.BlockSpec((1, D), rep),
        pl.BlockSpec((1, D), rep),
        pl.BlockSpec((1, D), rep),
        pl.BlockSpec((1, D), rep),
        pl.BlockSpec((1, D), rep),
        pl.BlockSpec((1, D), rep),
        pl.BlockSpec((1, D), rep),
        pl.BlockSpec((D, 2 * D + 32), rep),
        pl.BlockSpec((1, D), rep),
        pl.BlockSpec((32, 1), rep),
        pl.BlockSpec((1, 32), rep),
        pl.BlockSpec((128, 2 * D + 32), rep),
        pl.BlockSpec((1, 128), rep),
        pl.BlockSpec((32, 1), rep),
        pl.BlockSpec((1, 32), rep),
        pl.BlockSpec((128, D), rep),
        pl.BlockSpec((1, 128), rep),
        pl.BlockSpec((D, 2 * D), rep),
        pl.BlockSpec((1, D), rep),
        pl.BlockSpec((128, D), rep),
        pl.BlockSpec((1, 128), rep),
        pl.BlockSpec((3, 128), rep),
        pl.BlockSpec((1, 3), rep),
        pl.BlockSpec((3, 128), rep),
        pl.BlockSpec((1, 3), rep),
        pl.BlockSpec((2, 128), rep),
        pl.BlockSpec((1, 2), rep),
        pl.BlockSpec((1, 1), rep),
    ]
    out_specs = [
        pl.BlockSpec((BLOCK, 3), row),
        pl.BlockSpec((BLOCK // 128, 128), row),
        pl.BlockSpec((BLOCK, 3), row),
        pl.BlockSpec((BLOCK, 3), row),
        pl.BlockSpec((BLOCK, 2), row),
    ]
    out_shape = [
        jax.ShapeDtypeStruct((B, 3), f32),
        jax.ShapeDtypeStruct((B // 128, 128), f32),
        jax.ShapeDtypeStruct((B, 3), f32),
        jax.ShapeDtypeStruct((B, 3), f32),
        jax.ShapeDtypeStruct((B, 2), f32),
    ]
    scratch_shapes = [
        pltpu.VMEM((2 * D, D), bf16),
        pltpu.VMEM((2 * D, D), bf16),
        pltpu.VMEM((1, D), f32),
        pltpu.VMEM((1, D), f32),
        pltpu.VMEM((1, 384), f32),
        pltpu.VMEM((1, 384), f32),
        pltpu.VMEM((384, D), bf16),
        pltpu.VMEM((384, D), bf16),
        pltpu.VMEM((128, D), bf16),
        pltpu.VMEM((128, D), bf16),
    ]
    outs = pl.pallas_call(
        _kernel_body,
        grid=grid,
        in_specs=in_specs,
        out_specs=out_specs,
        out_shape=out_shape,
        scratch_shapes=scratch_shapes,
        compiler_params=pltpu.CompilerParams(
            dimension_semantics=("arbitrary",)),
    )(s_t, s_i, gds.reshape(B // 128, 128),
      p['ca_kvpi_w'], p['ca_opt_w'], p['ca_kvpt_w'], p['ca_opi_w'],
      p['ca_kvpi_b'].reshape(2, D)[1:], p['ca_opt_b'][None],
      p['ca_kvpt_b'].reshape(2, D)[1:], p['ca_opi_b'][None],
      p['ca_lnt_g'][None], p['ca_lnt_b'][None],
      p['ca_lni_g'][None], p['ca_lni_b'][None],
      p['cb_c0_w'], p['cb_c0_b'][None],
      p['cb_gds_w'], p['cb_gds_b'][None],
      p['sh_h0_w'], p['sh_h0_b'][None],
      p['sh_gds_w'], p['sh_gds_b'][None],
      p['cb_c1_w'], p['cb_c1_b'][None],
      p['nb_m0_w'], p['nb_m0_b'][None],
      p['nb_m1_w'], p['nb_m1_b'][None],
      p['cb_c2_w'], p['cb_c2_b'][None],
      p['nb_m2_w'], p['nb_m2_b'][None],
      p['sh_h1_w'], p['sh_h1_b'][None],
      p['log_threshold'].reshape(1, 1))
    logits, routing, normal, conflict, sarcasm = outs
    return logits, routing.reshape(B), normal, conflict, sarcasm


def kernel(s_t, s_i, gds, params):
    return _run(s_t, s_i, gds, params)


# speedup vs baseline: 1.6853x; 1.6149x over previous
import jax
import jax.numpy as jnp
from jax.experimental import pallas as pl
from jax.experimental.pallas import tpu as pltpu

D = 256
TEMPERATURE = 10.0
BLOCK = 2048

_NT = (((1,), (1,)), ((), ()))


def _dnt(a, b):
    return jax.lax.dot_general(a, b, dimension_numbers=_NT,
                               preferred_element_type=jnp.float32)


def _gelu_exact(x):
    return 0.5 * x * (1.0 + jax.lax.erf(x * 0.7071067811865476))


def _ln(x, g, b, eps=1e-5):
    m = jnp.mean(x, axis=-1, keepdims=True)
    c = x - m
    v = jnp.mean(c * c, axis=-1, keepdims=True)
    return c * jax.lax.rsqrt(v + eps) * g + b


def _kernel_body(xt_ref, xi_ref, g_ref,
                 wvi_ref, wot_ref, wvt_ref, woi_ref,
                 bvi_ref, bot_ref, bvt_ref, boi_ref,
                 lntg_ref, lntb_ref, lnig_ref, lnib_ref,
                 c0_ref, c0b_ref, gwc_ref, gbc_ref,
                 sh0_ref, sh0b_ref, gws_ref, gbs_ref,
                 c1_ref, c1b_ref,
                 n0_ref, n0b_ref, n1_ref, n1b_ref,
                 c2_ref, c2b_ref, m2_ref, m2b_ref, sh1_ref, sh1b_ref,
                 lt_ref,
                 logits_ref, routing_ref, normal_ref, conflict_ref,
                 sarcasm_ref,
                 s_xtw, s_xiw, s_bt, s_bi, s_ua, s_ba,
                 s_wt, s_wi, s_c1, s_n1):
    f32 = jnp.float32
    bf16 = jnp.bfloat16

    @pl.when(pl.program_id(0) == 0)
    def _fold():
        wot = wot_ref[:].astype(bf16)
        wvi = wvi_ref[:].astype(bf16)
        woi = woi_ref[:].astype(bf16)
        wvt = wvt_ref[:].astype(bf16)
        s_xiw[0:D, :] = jnp.dot(wot, wvi, preferred_element_type=f32).astype(bf16)
        s_xiw[D:, :] = n0_ref[:, D:].astype(bf16)
        s_xtw[0:D, :] = jnp.dot(woi, wvt, preferred_element_type=f32).astype(bf16)
        s_xtw[D:, :] = n0_ref[:, :D].astype(bf16)
        s_bt[:] = _dnt(bvi_ref[:], wot_ref[:]) + bot_ref[:]
        s_bi[:] = _dnt(bvt_ref[:], woi_ref[:]) + boi_ref[:]
        wgc = c0_ref[:, 2 * D:2 * D + 32]
        wgs = sh0_ref[:, 2 * D:2 * D + 32]
        s_ua[:, 0:D] = jnp.reshape(
            jnp.dot(wgc, gwc_ref[:], preferred_element_type=f32), (1, D))
        s_ua[:, D:] = jnp.reshape(
            jnp.dot(wgs, gws_ref[:], preferred_element_type=f32), (1, 128))
        s_ba[:, 0:D] = _dnt(gbc_ref[:], wgc) + c0b_ref[:]
        s_ba[:, D:] = _dnt(gbs_ref[:], wgs) + sh0b_ref[:]
        s_wt[0:D, :] = c0_ref[:, :D].astype(bf16)
        s_wt[D:, :] = sh0_ref[:, :D].astype(bf16)
        s_wi[0:D, :] = c0_ref[:, D:2 * D].astype(bf16)
        s_wi[D:, :] = sh0_ref[:, D:2 * D].astype(bf16)
        s_c1[:] = c1_ref[:].astype(bf16)
        s_n1[:] = n1_ref[:].astype(bf16)

    xt = xt_ref[:]
    xi = xi_ref[:]
    g2 = g_ref[:]
    gt = jnp.transpose(g2)
    g = jnp.concatenate(
        [gt[:, r:r + 1] for r in range(g2.shape[0])], axis=0)
    xtb = xt.astype(bf16)
    xib = xi.astype(bf16)
    p_t = _dnt(xtb, s_xtw[:])
    p_i = _dnt(xib, s_xiw[:])
    t_refv = _ln(xt + (p_i[:, :D] + s_bt[:]), lntg_ref[:], lntb_ref[:])
    i_refv = _ln(xi + (p_t[:, :D] + s_bi[:]), lnig_ref[:], lnib_ref[:])
    trb = t_refv.astype(bf16)
    irb = i_refv.astype(bf16)
    q = _dnt(trb, s_wt[:]) + _dnt(irb, s_wi[:]) + g * s_ua[:] + s_ba[:]
    qa = _gelu_exact(q.astype(bf16))
    h0b = qa[:, :D]
    hsb = qa[:, D:]
    h1b = _gelu_exact((_dnt(h0b, s_c1[:]) + c1b_ref[:]).astype(bf16))
    n_pre = p_t[:, D:] + p_i[:, D:] + n0b_ref[:]
    n0b_ = _gelu_exact(n_pre.astype(bf16))
    n1b_ = _gelu_exact((_dnt(n0b_, s_n1[:]) + n1b_ref[:]).astype(bf16))
    conflict = _dnt(c2_ref[:].astype(bf16), h1b) + jnp.transpose(c2b_ref[:])
    normal = _dnt(m2_ref[:].astype(bf16), n1b_) + jnp.transpose(m2b_ref[:])
    sarcasm = _dnt(sh1_ref[:].astype(bf16), hsb) + jnp.transpose(sh1b_ref[:])
    tau = jax.nn.sigmoid(lt_ref[:])
    g_row = jnp.concatenate(
        [g2[r:r + 1, :] for r in range(g2.shape[0])], axis=1)
    gate = jax.nn.sigmoid((g_row - tau) * TEMPERATURE)
    logits_ref[:] = gate * conflict + (1.0 - gate) * normal
    routing_ref[:] = (g2 > tau).astype(f32)
    normal_ref[:] = normal
    conflict_ref[:] = conflict
    sarcasm_ref[:] = sarcasm


@jax.jit
def _run(s_t, s_i, gds, params):
    f32 = jnp.float32
    bf16 = jnp.bfloat16
    p = params
    B = s_t.shape[0]
    grid = (B // BLOCK,)
    row = lambda i: (i, 0)
    rep = lambda i: (0, 0)
    vhalf = lambda i: (1, 0)

    in_specs = [
        pl.BlockSpec((BLOCK, D), row),
        pl.BlockSpec((BLOCK, D), row),
        pl.BlockSpec((BLOCK // 128, 128), row),
        pl.BlockSpec((D, D), vhalf),
        pl.BlockSpec((D, D), rep),
        pl.BlockSpec((D, D), vhalf),
        pl.BlockSpec((D, D), rep),
        pl.BlockSpec((1, D), rep),
        pl.BlockSpec((1, D), rep),
        pl.BlockSpec((1, D), rep),
        pl.BlockSpec((1, D), rep),
        pl.BlockSpec((1, D), rep),
        pl.BlockSpec((1, D), rep),
        pl.BlockSpec((1, D), rep),
        pl.BlockSpec((1, D), rep),
        pl.BlockSpec((D, 2 * D + 32), rep),
        pl.BlockSpec((1, D), rep),
        pl.BlockSpec((32, 1), rep),
        pl.BlockSpec((1, 32), rep),
        pl.BlockSpec((128, 2 * D + 32), rep),
        pl.BlockSpec((1, 128), rep),
        pl.BlockSpec((32, 1), rep),
        pl.BlockSpec((1, 32), rep),
        pl.BlockSpec((128, D), rep),
        pl.BlockSpec((1, 128), rep),
        pl.BlockSpec((D, 2 * D), rep),
        pl.BlockSpec((1, D), rep),
        pl.BlockSpec((128, D), rep),
        pl.BlockSpec((1, 128), rep),
        pl.BlockSpec((3, 128), rep),
        pl.BlockSpec((1, 3), rep),
        pl.BlockSpec((3, 128), rep),
        pl.BlockSpec((1, 3), rep),
        pl.BlockSpec((2, 128), rep),
        pl.BlockSpec((1, 2), rep),
        pl.BlockSpec((1, 1), rep),
    ]
    col = lambda i: (0, i)
    out_specs = [
        pl.BlockSpec((3, BLOCK), col),
        pl.BlockSpec((BLOCK // 128, 128), row),
        pl.BlockSpec((3, BLOCK), col),
        pl.BlockSpec((3, BLOCK), col),
        pl.BlockSpec((2, BLOCK), col),
    ]
    out_shape = [
        jax.ShapeDtypeStruct((3, B), f32),
        jax.ShapeDtypeStruct((B // 128, 128), f32),
        jax.ShapeDtypeStruct((3, B), f32),
        jax.ShapeDtypeStruct((3, B), f32),
        jax.ShapeDtypeStruct((2, B), f32),
    ]
    scratch_shapes = [
        pltpu.VMEM((2 * D, D), bf16),
        pltpu.VMEM((2 * D, D), bf16),
        pltpu.VMEM((1, D), f32),
        pltpu.VMEM((1, D), f32),
        pltpu.VMEM((1, 384), f32),
        pltpu.VMEM((1, 384), f32),
        pltpu.VMEM((384, D), bf16),
        pltpu.VMEM((384, D), bf16),
        pltpu.VMEM((128, D), bf16),
        pltpu.VMEM((128, D), bf16),
    ]
    outs = pl.pallas_call(
        _kernel_body,
        grid=grid,
        in_specs=in_specs,
        out_specs=out_specs,
        out_shape=out_shape,
        scratch_shapes=scratch_shapes,
        compiler_params=pltpu.CompilerParams(
            dimension_semantics=("arbitrary",)),
    )(s_t, s_i, gds.reshape(B // 128, 128),
      p['ca_kvpi_w'], p['ca_opt_w'], p['ca_kvpt_w'], p['ca_opi_w'],
      p['ca_kvpi_b'].reshape(2, D)[1:], p['ca_opt_b'][None],
      p['ca_kvpt_b'].reshape(2, D)[1:], p['ca_opi_b'][None],
      p['ca_lnt_g'][None], p['ca_lnt_b'][None],
      p['ca_lni_g'][None], p['ca_lni_b'][None],
      p['cb_c0_w'], p['cb_c0_b'][None],
      p['cb_gds_w'], p['cb_gds_b'][None],
      p['sh_h0_w'], p['sh_h0_b'][None],
      p['sh_gds_w'], p['sh_gds_b'][None],
      p['cb_c1_w'], p['cb_c1_b'][None],
      p['nb_m0_w'], p['nb_m0_b'][None],
      p['nb_m1_w'], p['nb_m1_b'][None],
      p['cb_c2_w'], p['cb_c2_b'][None],
      p['nb_m2_w'], p['nb_m2_b'][None],
      p['sh_h1_w'], p['sh_h1_b'][None],
      p['log_threshold'].reshape(1, 1))
    logits, routing, normal, conflict, sarcasm = outs
    return (logits.T, routing.reshape(B), normal.T, conflict.T, sarcasm.T)


def kernel(s_t, s_i, gds, params):
    return _run(s_t, s_i, gds, params)
